# Initial kernel scaffold; baseline (speedup 1.0000x reference)
import jax
import jax.numpy as jnp
from jax.experimental import pallas as pl


def kernel(batch_data, table):
    raise NotImplementedError("write your pallas kernel here")


# speedup vs baseline: 1.1563x; 1.1563x over previous
import functools

import jax
import jax.numpy as jnp
from jax import lax
from jax.experimental import pallas as pl
from jax.experimental.pallas import tpu as pltpu
from jax.experimental.pallas import tpu_sc as plsc

EMB = 32
NSETS = 9
BATCH = 4096
HIST = 50
ROWS_PER_SET = BATCH * HIST
NC, NS = 2, 16
NW = NC * NS
ROWS_PER_WORKER = ROWS_PER_SET // NW
GRP = 128
CHUNK = 1280
GROUPS = CHUNK // GRP
NCHUNK = ROWS_PER_WORKER // CHUNK

_mesh = plsc.VectorSubcoreMesh(core_axis_name="c", subcore_axis_name="s")


@functools.partial(
    pl.kernel,
    mesh=_mesh,
    out_type=tuple(
        jax.ShapeDtypeStruct((ROWS_PER_SET, EMB), jnp.float32)
        for _ in range(NSETS)
    ),
    scratch_types=[
        pltpu.VMEM((CHUNK,), jnp.int32),
        pltpu.VMEM((CHUNK, EMB), jnp.float32),
        pltpu.SemaphoreType.DMA,
    ],
    compiler_params=pltpu.CompilerParams(use_tc_tiling_on_sc=False),
)
def _emb_gather(idx_hbm, table_hbm, *refs):
    outs = refs[:NSETS]
    idx_v, rows_v, sem = refs[NSETS:]
    wid = lax.axis_index("s") * NC + lax.axis_index("c")

    for t in range(NSETS):
        out = outs[t]

        def chunk_body(c, carry, out=out, t=t):
            rbase = pl.multiple_of(wid * ROWS_PER_WORKER + c * CHUNK, CHUNK)
            ibase = pl.multiple_of(
                t * ROWS_PER_SET + wid * ROWS_PER_WORKER + c * CHUNK, CHUNK
            )
            pltpu.sync_copy(idx_hbm.at[pl.ds(ibase, CHUNK)], idx_v)

            def fire(j, carry2):
                off = pl.multiple_of(j * GRP, GRP)
                pltpu.async_copy(
                    table_hbm.at[idx_v.at[pl.ds(off, GRP)]],
                    rows_v.at[pl.ds(off, GRP)],
                    sem,
                )
                return carry2

            lax.fori_loop(0, GROUPS, fire, 0)
            pltpu.make_async_copy(
                table_hbm.at[pl.ds(0, CHUNK)], rows_v, sem
            ).wait()
            pltpu.sync_copy(rows_v, out.at[pl.ds(rbase, CHUNK)])
            return carry

        lax.fori_loop(0, NCHUNK, chunk_body, 0)


def kernel(batch_data, table):
    idx = batch_data.reshape(NSETS * ROWS_PER_SET)
    outs = _emb_gather(idx, table)
    return tuple(o.reshape(BATCH, HIST, EMB) for o in outs)


# speedup vs baseline: 1.1802x; 1.0207x over previous
import functools

import jax
import jax.numpy as jnp
from jax import lax
from jax.experimental import pallas as pl
from jax.experimental.pallas import tpu as pltpu
from jax.experimental.pallas import tpu_sc as plsc

EMB = 32
NSETS = 9
BATCH = 4096
HIST = 50
ROWS_PER_SET = BATCH * HIST
NC, NS = 2, 16
NW = NC * NS
ROWS_PER_WORKER = ROWS_PER_SET // NW
GRP = 128
CHUNK = 1280
GROUPS = CHUNK // GRP
NCHUNK = ROWS_PER_WORKER // CHUNK
NBUF = 3

_mesh = plsc.VectorSubcoreMesh(core_axis_name="c", subcore_axis_name="s")


@functools.partial(
    pl.kernel,
    mesh=_mesh,
    out_type=tuple(
        jax.ShapeDtypeStruct((ROWS_PER_SET, EMB), jnp.float32)
        for _ in range(NSETS)
    ),
    scratch_types=(
        [pltpu.VMEM((CHUNK,), jnp.int32) for _ in range(NBUF)]
        + [pltpu.VMEM((CHUNK, EMB), jnp.float32) for _ in range(NBUF)]
        + [pltpu.SemaphoreType.DMA for _ in range(2 * NBUF)]
    ),
    compiler_params=pltpu.CompilerParams(use_tc_tiling_on_sc=False),
)
def _emb_gather(idx_hbm, table_hbm, *refs):
    outs = refs[:NSETS]
    idx_v = refs[NSETS : NSETS + NBUF]
    rows_v = refs[NSETS + NBUF : NSETS + 2 * NBUF]
    gsem = refs[NSETS + 2 * NBUF : NSETS + 2 * NBUF + NBUF]
    osem = refs[NSETS + 3 * NBUF :]
    wid = lax.axis_index("s") * NC + lax.axis_index("c")

    chunks = [(t, c) for t in range(NSETS) for c in range(NCHUNK)]
    pending_store = [None] * NBUF

    def fire_gathers(b):
        def fire(j, carry):
            off = pl.multiple_of(j * GRP, GRP)
            pltpu.async_copy(
                table_hbm.at[idx_v[b].at[pl.ds(off, GRP)]],
                rows_v[b].at[pl.ds(off, GRP)],
                gsem[b],
            )
            return carry

        lax.fori_loop(0, GROUPS, fire, 0)

    def drain_gathers(b):
        pltpu.make_async_copy(
            table_hbm.at[pl.ds(0, CHUNK)], rows_v[b], gsem[b]
        ).wait()

    for g, (t, c) in enumerate(chunks):
        b = g % NBUF
        if pending_store[b] is not None:
            pending_store[b].wait()
            pending_store[b] = None
        ibase = pl.multiple_of(
            t * ROWS_PER_SET + wid * ROWS_PER_WORKER + c * CHUNK, CHUNK
        )
        pltpu.sync_copy(idx_hbm.at[pl.ds(ibase, CHUNK)], idx_v[b])
        fire_gathers(b)
        if g >= 1:
            tp, cp = chunks[g - 1]
            bp = (g - 1) % NBUF
            drain_gathers(bp)
            rbase = pl.multiple_of(
                wid * ROWS_PER_WORKER + cp * CHUNK, CHUNK
            )
            pending_store[bp] = pltpu.async_copy(
                rows_v[bp], outs[tp].at[pl.ds(rbase, CHUNK)], osem[bp]
            )

    tl, cl = chunks[-1]
    bl = (len(chunks) - 1) % NBUF
    drain_gathers(bl)
    rbase = pl.multiple_of(wid * ROWS_PER_WORKER + cl * CHUNK, CHUNK)
    pending_store[bl] = pltpu.async_copy(
        rows_v[bl], outs[tl].at[pl.ds(rbase, CHUNK)], osem[bl]
    )
    for h in pending_store:
        if h is not None:
            h.wait()


def kernel(batch_data, table):
    idx = batch_data.reshape(NSETS * ROWS_PER_SET)
    outs = _emb_gather(idx, table)
    return tuple(o.reshape(BATCH, HIST, EMB) for o in outs)


# speedup vs baseline: 1.5593x; 1.3212x over previous
import functools

import jax
import jax.numpy as jnp
from jax import lax
from jax.experimental import pallas as pl
from jax.experimental.pallas import tpu as pltpu
from jax.experimental.pallas import tpu_sc as plsc

EMB = 32
NSETS = 9
BATCH = 4096
HIST = 50
ROWS_PER_SET = BATCH * HIST
NC, NS = 2, 16
NW = NC * NS
SLAB = 128 * HIST

_mesh = plsc.VectorSubcoreMesh(core_axis_name="c", subcore_axis_name="s")


@functools.partial(
    pl.kernel,
    mesh=_mesh,
    out_type=tuple(
        jax.ShapeDtypeStruct((HIST, EMB // 8, BATCH // 128, 8, 128), jnp.float32)
        for _ in range(NSETS)
    ),
    scratch_types=(
        [pltpu.VMEM((SLAB,), jnp.int32)]
        + [pltpu.VMEM((128,), jnp.int32) for _ in range(2)]
        + [pltpu.VMEM((128, EMB), jnp.float32) for _ in range(2)]
        + [pltpu.VMEM((EMB // 8, 8, 128), jnp.float32) for _ in range(2)]
        + [pltpu.SemaphoreType.DMA for _ in range(4)]
    ),
    compiler_params=pltpu.CompilerParams(
        use_tc_tiling_on_sc=False, needs_layout_passes=False
    ),
)
def _emb_gather(idx_hbm, table_hbm, *refs):
    outs = refs[:NSETS]
    slab = refs[NSETS]
    gidx = refs[NSETS + 1 : NSETS + 3]
    grows = refs[NSETS + 3 : NSETS + 5]
    trans = refs[NSETS + 5 : NSETS + 7]
    gsem = refs[NSETS + 7 : NSETS + 9]
    osem = refs[NSETS + 9 : NSETS + 11]
    wid = lax.axis_index("s") * NC + lax.axis_index("c")

    iota = lax.iota(jnp.int32, 16)
    v50 = iota * 50
    rows_k = [iota + 16 * k for k in range(8)]

    def extract_and_fire(h, p):
        for k in range(8):
            addr = v50 + (800 * k + h)
            vals = plsc.load_gather(slab, [addr])
            gidx[p][pl.ds(16 * k, 16)] = vals
        pltpu.async_copy(table_hbm.at[gidx[p]], grows[p], gsem[p])

    def drain_gather(p):
        pltpu.make_async_copy(
            table_hbm.at[pl.ds(0, 128)], grows[p], gsem[p]
        ).wait()

    def drain_stores(p, t):
        pltpu.make_async_copy(
            outs[t].at[0, pl.ds(0, EMB // 8), 0], trans[p], osem[p]
        ).wait()

    def transpose(p):
        def kbody(k, carry):
            rows = iota + 16 * k
            off = pl.multiple_of(16 * k, 16)
            for d in range(EMB):
                cols = jnp.full((16,), d, jnp.int32)
                vals = plsc.load_gather(grows[p], [rows, cols])
                trans[p][d // 8, d % 8, pl.ds(off, 16)] = vals
            return carry

        lax.fori_loop(0, 8, kbody, 0)

    def fire_stores(h, p, t):
        pltpu.async_copy(
            trans[p], outs[t].at[h, pl.ds(0, EMB // 8), wid], osem[p]
        )

    for t in range(NSETS):
        base = pl.multiple_of(t * ROWS_PER_SET + wid * SLAB, SLAB)
        pltpu.sync_copy(idx_hbm.at[pl.ds(base, SLAB)], slab)
        extract_and_fire(jnp.int32(0), 0)

        def h2body(h2, carry, t=t):
            for p in range(2):
                h = 2 * h2 + p
                if p == 0:
                    extract_and_fire(h + 1, 1)
                else:

                    @pl.when(h2 < 24)
                    def _():
                        extract_and_fire(h + 1, 0)

                drain_gather(p)
                if t == 0:

                    @pl.when(h2 > 0)
                    def _():
                        drain_stores(p, t)

                else:
                    drain_stores(p, t)
                transpose(p)
                fire_stores(h, p, t)
            return carry

        lax.fori_loop(0, HIST // 2, h2body, 0)

    for p in range(2):
        drain_stores(p, NSETS - 1)


def kernel(batch_data, table):
    idx = batch_data.reshape(NSETS * ROWS_PER_SET)
    outs = _emb_gather(idx, table)
    return tuple(
        o.transpose(2, 4, 0, 1, 3).reshape(BATCH, HIST, EMB) for o in outs
    )


# speedup vs baseline: 2.4900x; 1.5969x over previous
import functools

import jax
import jax.numpy as jnp
from jax import lax
from jax.experimental import pallas as pl
from jax.experimental.pallas import tpu as pltpu
from jax.experimental.pallas import tpu_sc as plsc

EMB = 32
VOCAB_ROWS = 1000000
NSETS = 9
BATCH = 4096
HIST = 50
ROWS_PER_SET = BATCH * HIST
NC, NS = 2, 16
NW = NC * NS
SLAB = 128 * HIST

_mesh = plsc.VectorSubcoreMesh(core_axis_name="c", subcore_axis_name="s")


@functools.partial(
    pl.kernel,
    mesh=_mesh,
    out_type=tuple(
        jax.ShapeDtypeStruct((HIST, EMB // 8, BATCH // 128, 8, 128), jnp.float32)
        for _ in range(NSETS)
    ),
    scratch_types=(
        [pltpu.VMEM((SLAB,), jnp.int32)]
        + [pltpu.VMEM((128,), jnp.int32) for _ in range(2)]
        + [pltpu.VMEM((128, EMB), jnp.float32) for _ in range(2)]
        + [pltpu.VMEM((EMB, 129), jnp.float32) for _ in range(2)]
        + [pltpu.SemaphoreType.DMA for _ in range(4)]
    ),
    compiler_params=pltpu.CompilerParams(
        use_tc_tiling_on_sc=False, needs_layout_passes=False
    ),
)
def _emb_gather(idx_hbm, table_hbm, *refs):
    outs = refs[:NSETS]
    slab = refs[NSETS]
    gidx = refs[NSETS + 1 : NSETS + 3]
    grows = refs[NSETS + 3 : NSETS + 5]
    trans = refs[NSETS + 5 : NSETS + 7]
    gsem = refs[NSETS + 7 : NSETS + 9]
    osem = refs[NSETS + 9 : NSETS + 11]
    wid = lax.axis_index("s") * NC + lax.axis_index("c")

    iota = lax.iota(jnp.int32, 16)
    v50 = iota * 50
    rows2 = [iota, iota + 16]

    def extract_and_fire(h, p):
        vals = [
            plsc.load_gather(slab, [v50 + (800 * k + h)]) for k in range(8)
        ]
        for k in range(8):
            gidx[p][pl.ds(16 * k, 16)] = vals[k] * 2
        pltpu.async_copy(table_hbm.at[gidx[p]], grows[p], gsem[p])

    def drain_gather(p):
        pltpu.make_async_copy(
            table_hbm.at[pl.ds(0, 128)], grows[p], gsem[p]
        ).wait()

    def drain_stores(p, t):
        for dt in range(EMB // 8):
            pltpu.make_async_copy(
                outs[t].at[0, dt, 0],
                trans[p].at[pl.ds(8 * dt, 8), pl.ds(0, 128)],
                osem[p],
            ).wait()

    def transpose(p):
        def bbody(bc0, carry):
            base = bc0 * 8
            vals = [
                grows[p][base + j, pl.ds(16 * k2, 16)]
                for j in range(8)
                for k2 in range(2)
            ]
            for j in range(8):
                cols = jnp.full((16,), base + j, jnp.int32)
                for k2 in range(2):
                    plsc.store_scatter(
                        trans[p], [rows2[k2], cols], vals[2 * j + k2]
                    )
            return carry

        lax.fori_loop(0, 16, bbody, 0)

    def fire_stores(h, p, t):
        for dt in range(EMB // 8):
            pltpu.async_copy(
                trans[p].at[pl.ds(8 * dt, 8), pl.ds(0, 128)],
                outs[t].at[h, dt, wid],
                osem[p],
            )

    for t in range(NSETS):
        base = pl.multiple_of(t * ROWS_PER_SET + wid * SLAB, SLAB)
        pltpu.sync_copy(idx_hbm.at[pl.ds(base, SLAB)], slab)
        extract_and_fire(jnp.int32(0), 0)

        def h2body(h2, carry, t=t):
            for p in range(2):
                h = 2 * h2 + p
                if p == 0:
                    extract_and_fire(h + 1, 1)
                else:

                    @pl.when(h2 < 24)
                    def _():
                        extract_and_fire(h + 1, 0)

                drain_gather(p)
                if t == 0:

                    @pl.when(h2 > 0)
                    def _():
                        drain_stores(p, t)

                else:
                    drain_stores(p, t)
                transpose(p)
                fire_stores(h, p, t)
            return carry

        lax.fori_loop(0, HIST // 2, h2body, 0)

    for p in range(2):
        drain_stores(p, NSETS - 1)


def kernel(batch_data, table):
    idx = batch_data.reshape(NSETS * ROWS_PER_SET)
    tablep = jnp.pad(table, ((0, 0), (0, 32))).reshape(2 * VOCAB_ROWS, EMB)
    outs = _emb_gather(idx, tablep)
    return tuple(
        o.transpose(2, 4, 0, 1, 3).reshape(BATCH, HIST, EMB) for o in outs
    )


# speedup vs baseline: 3.6420x; 1.4626x over previous
import functools

import jax
import jax.numpy as jnp
from jax import lax
from jax.experimental import pallas as pl
from jax.experimental.pallas import tpu as pltpu
from jax.experimental.pallas import tpu_sc as plsc

EMB = 32
VOCAB_ROWS = 1000000
NSETS = 9
BATCH = 4096
HIST = 50
ROWS_PER_SET = BATCH * HIST
NC, NS = 2, 16
NW = NC * NS
SLAB = 128 * HIST

_mesh = plsc.VectorSubcoreMesh(core_axis_name="c", subcore_axis_name="s")


@functools.partial(
    pl.kernel,
    mesh=_mesh,
    out_type=tuple(
        jax.ShapeDtypeStruct((HIST, EMB // 8, BATCH // 128, 8, 128), jnp.float32)
        for _ in range(NSETS)
    ),
    scratch_types=(
        [pltpu.VMEM((SLAB,), jnp.int32)]
        + [pltpu.VMEM((128,), jnp.int32) for _ in range(2)]
        + [pltpu.VMEM((128, EMB), jnp.float32) for _ in range(2)]
        + [pltpu.VMEM((EMB, 129), jnp.float32) for _ in range(2)]
        + [pltpu.SemaphoreType.DMA for _ in range(4)]
    ),
    compiler_params=pltpu.CompilerParams(
        use_tc_tiling_on_sc=False, needs_layout_passes=False
    ),
)
def _emb_gather(idx_hbm, table_hbm, *refs):
    outs = refs[:NSETS]
    slab = refs[NSETS]
    gidx = refs[NSETS + 1 : NSETS + 3]
    grows = refs[NSETS + 3 : NSETS + 5]
    trans = refs[NSETS + 5 : NSETS + 7]
    gsem = refs[NSETS + 7 : NSETS + 9]
    osem = refs[NSETS + 9 : NSETS + 11]
    wid = lax.axis_index("s") * NC + lax.axis_index("c")

    iota = lax.iota(jnp.int32, 16)
    v50 = iota * 50
    rows2 = [iota, iota + 16]

    def extract_and_fire(h, p):
        vals = [
            plsc.load_gather(slab, [v50 + (800 * k + h)]) for k in range(8)
        ]
        for k in range(8):
            gidx[p][pl.ds(16 * k, 16)] = vals[k] * 4
        pltpu.async_copy(table_hbm.at[gidx[p]], grows[p], gsem[p])

    def drain_gather(p):
        pltpu.make_async_copy(
            table_hbm.at[pl.ds(0, 128)], grows[p], gsem[p]
        ).wait()

    def drain_stores(p, t):
        for dt in range(EMB // 8):
            pltpu.make_async_copy(
                outs[t].at[0, dt, 0],
                trans[p].at[pl.ds(8 * dt, 8), pl.ds(0, 128)],
                osem[p],
            ).wait()

    def transpose(p):
        def bbody(bc0, carry):
            base = bc0 * 8
            vals = [
                grows[p][base + j, pl.ds(16 * k2, 16)]
                for j in range(8)
                for k2 in range(2)
            ]
            for j in range(8):
                cols = jnp.full((16,), base + j, jnp.int32)
                for k2 in range(2):
                    plsc.store_scatter(
                        trans[p], [rows2[k2], cols], vals[2 * j + k2]
                    )
            return carry

        lax.fori_loop(0, 16, bbody, 0)

    def fire_stores(h, p, t):
        for dt in range(EMB // 8):
            pltpu.async_copy(
                trans[p].at[pl.ds(8 * dt, 8), pl.ds(0, 128)],
                outs[t].at[h, dt, wid],
                osem[p],
            )

    for t in range(NSETS):
        base = pl.multiple_of(t * ROWS_PER_SET + wid * SLAB, SLAB)
        pltpu.sync_copy(idx_hbm.at[pl.ds(base, SLAB)], slab)
        extract_and_fire(jnp.int32(0), 0)

        def h2body(h2, carry, t=t):
            for p in range(2):
                h = 2 * h2 + p
                if p == 0:
                    extract_and_fire(h + 1, 1)
                else:

                    @pl.when(h2 < 24)
                    def _():
                        extract_and_fire(h + 1, 0)

                drain_gather(p)
                if t == 0:

                    @pl.when(h2 > 0)
                    def _():
                        drain_stores(p, t)

                else:
                    drain_stores(p, t)
                transpose(p)
                fire_stores(h, p, t)
            return carry

        lax.fori_loop(0, HIST // 2, h2body, 0)

    for p in range(2):
        drain_stores(p, NSETS - 1)


def kernel(batch_data, table):
    idx = batch_data.reshape(NSETS * ROWS_PER_SET)
    tablep = jnp.pad(table, ((0, 0), (0, 96))).reshape(4 * VOCAB_ROWS, EMB)
    outs = _emb_gather(idx, tablep)
    return tuple(
        o.transpose(2, 4, 0, 1, 3).reshape(BATCH, HIST, EMB) for o in outs
    )
